# Initial kernel scaffold; baseline (speedup 1.0000x reference)
import jax
import jax.numpy as jnp
from jax.experimental import pallas as pl


def kernel(qkv, cls_embedding, W_cls):
    raise NotImplementedError("write your pallas kernel here")


# speedup vs baseline: 2.0251x; 2.0251x over previous
import math

import jax
import jax.numpy as jnp
from jax.experimental import pallas as pl
from jax.experimental.pallas import tpu as pltpu


def _pab_kernel(q_ref, kv_ref, eq_ref, ef_ref, w_ref, o_ref):
    ch = q_ref.shape[1]
    scale = 1.0 / math.sqrt(math.sqrt(ch))
    w_q = w_ref[0:ch, :]
    w_k = w_ref[ch:2 * ch, :]
    w_v = w_ref[2 * ch:3 * ch, :]
    eq = eq_ref[0]
    ef = ef_ref[0]
    q = q_ref[0]
    k = kv_ref[0, 0:ch, :]
    v = kv_ref[0, ch:2 * ch, :]
    dn_e = (((1,), (1,)), ((), ()))
    q_null = jax.lax.dot_general(w_q, eq, dn_e, preferred_element_type=jnp.float32)
    k_null = jax.lax.dot_general(w_k, ef, dn_e, preferred_element_type=jnp.float32)
    v_null = jax.lax.dot_general(w_v, ef, dn_e, preferred_element_type=jnp.float32)
    qe = (q + q_null) * scale
    ke = (k + k_null) * scale
    ve = v + v_null
    logits = jax.lax.dot_general(
        qe, ke, (((0,), (0,)), ((), ())), preferred_element_type=jnp.float32)
    p = jax.nn.softmax(logits, axis=-1)
    a = jax.lax.dot_general(
        ve, p, (((1,), (1,)), ((), ())), preferred_element_type=jnp.float32)
    o_ref[0] = a


def kernel(qkv, cls_embedding, W_cls):
    bs, width, T = qkv.shape
    n_heads = 16
    ch = width // (3 * n_heads)
    B = bs * n_heads
    E = cls_embedding.shape[2]
    qkv_r = qkv.reshape(B, 3 * ch, T)
    q = qkv_r[:, :ch, :]
    kv = qkv_r[:, ch:, :]
    Tq = 512
    nq = T // Tq
    out = pl.pallas_call(
        _pab_kernel,
        grid=(B, nq),
        in_specs=[
            pl.BlockSpec((1, ch, Tq), lambda b, i: (b, 0, i)),
            pl.BlockSpec((1, 2 * ch, T), lambda b, i: (b, 0, 0)),
            pl.BlockSpec((1, Tq, E), lambda b, i: (b, i, 0)),
            pl.BlockSpec((1, T, E), lambda b, i: (b, 0, 0)),
            pl.BlockSpec((3 * ch, E), lambda b, i: (0, 0)),
        ],
        out_specs=pl.BlockSpec((1, ch, Tq), lambda b, i: (b, 0, i)),
        out_shape=jax.ShapeDtypeStruct((B, ch, T), qkv.dtype),
    )(q, kv, cls_embedding, cls_embedding, W_cls)
    return out.reshape(bs, n_heads * ch, T)


# speedup vs baseline: 2.5944x; 1.2811x over previous
import math

import jax
import jax.numpy as jnp
from jax.experimental import pallas as pl
from jax.experimental.pallas import tpu as pltpu


def _pab_kernel(q_ref, kv_ref, e_ref, w_ref, o_ref):
    ch = q_ref.shape[1]
    T = q_ref.shape[2]
    scale2 = 1.0 / math.sqrt(ch)
    w_q = w_ref[0:ch, :]
    w_k = w_ref[ch:2 * ch, :]
    w_v = w_ref[2 * ch:3 * ch, :]
    e = e_ref[0]
    q = q_ref[0]
    k = kv_ref[0, 0:ch, :]
    v = kv_ref[0, ch:2 * ch, :]
    dn_e = (((1,), (1,)), ((), ()))
    q_null = jax.lax.dot_general(w_q, e, dn_e, preferred_element_type=jnp.float32)
    k_null = jax.lax.dot_general(w_k, e, dn_e, preferred_element_type=jnp.float32)
    v_null = jax.lax.dot_general(w_v, e, dn_e, preferred_element_type=jnp.float32)
    qe = (q + q_null) * scale2
    ke = k + k_null
    ve = v + v_null
    logits = jax.lax.dot_general(
        qe, ke, (((0,), (0,)), ((), ())), preferred_element_type=jnp.float32)
    ew = jnp.exp(logits)
    a = jax.lax.dot_general(
        ve, ew, (((1,), (1,)), ((), ())), preferred_element_type=jnp.float32)
    ones = jnp.ones((8, T), dtype=jnp.float32)
    sums = jax.lax.dot_general(
        ones, ew, (((1,), (1,)), ((), ())), preferred_element_type=jnp.float32)
    o_ref[0] = a / sums[0:1, :]


def kernel(qkv, cls_embedding, W_cls):
    bs, width, T = qkv.shape
    n_heads = 16
    ch = width // (3 * n_heads)
    B = bs * n_heads
    E = cls_embedding.shape[2]
    qkv_r = qkv.reshape(B, 3 * ch, T)
    q = qkv_r[:, :ch, :]
    kv = qkv_r[:, ch:, :]
    out = pl.pallas_call(
        _pab_kernel,
        grid=(B,),
        in_specs=[
            pl.BlockSpec((1, ch, T), lambda b: (b, 0, 0)),
            pl.BlockSpec((1, 2 * ch, T), lambda b: (b, 0, 0)),
            pl.BlockSpec((1, T, E), lambda b: (b, 0, 0)),
            pl.BlockSpec((3 * ch, E), lambda b: (0, 0)),
        ],
        out_specs=pl.BlockSpec((1, ch, T), lambda b: (b, 0, 0)),
        out_shape=jax.ShapeDtypeStruct((B, ch, T), qkv.dtype),
    )(q, kv, cls_embedding, W_cls)
    return out.reshape(bs, n_heads * ch, T)


# speedup vs baseline: 2.5984x; 1.0016x over previous
import math

import jax
import jax.numpy as jnp
from jax.experimental import pallas as pl
from jax.experimental.pallas import tpu as pltpu


def _pab_kernel(q_ref, kv_ref, e_ref, w_ref, o_ref):
    ch = q_ref.shape[1]
    T = q_ref.shape[2]
    scale = 1.0 / math.sqrt(math.sqrt(ch))
    w_q = w_ref[0:ch, :]
    w_k = w_ref[ch:2 * ch, :]
    w_v = w_ref[2 * ch:3 * ch, :]
    e = e_ref[0]
    q = q_ref[0]
    k = kv_ref[0, 0:ch, :]
    v = kv_ref[0, ch:2 * ch, :]
    dn_e = (((1,), (1,)), ((), ()))
    q_null = jax.lax.dot_general(w_q, e, dn_e, preferred_element_type=jnp.float32)
    k_null = jax.lax.dot_general(w_k, e, dn_e, preferred_element_type=jnp.float32)
    v_null = jax.lax.dot_general(w_v, e, dn_e, preferred_element_type=jnp.float32)
    qe = (q + q_null) * scale
    ke = (k + k_null) * scale
    ve = v + v_null
    logits = jax.lax.dot_general(
        qe, ke, (((0,), (0,)), ((), ())), preferred_element_type=jnp.float32)
    ew = jnp.exp(logits)
    a = jax.lax.dot_general(
        ve, ew, (((1,), (1,)), ((), ())), preferred_element_type=jnp.float32)
    ones = jnp.ones((8, T), dtype=jnp.float32)
    sums = jax.lax.dot_general(
        ones, ew, (((1,), (1,)), ((), ())), preferred_element_type=jnp.float32)
    o_ref[0] = a / sums[0:1, :]


def kernel(qkv, cls_embedding, W_cls):
    bs, width, T = qkv.shape
    n_heads = 16
    ch = width // (3 * n_heads)
    B = bs * n_heads
    E = cls_embedding.shape[2]
    qkv_r = qkv.reshape(B, 3 * ch, T)
    q = qkv_r[:, :ch, :]
    kv = qkv_r[:, ch:, :]
    out = pl.pallas_call(
        _pab_kernel,
        grid=(B,),
        in_specs=[
            pl.BlockSpec((1, ch, T), lambda b: (b, 0, 0)),
            pl.BlockSpec((1, 2 * ch, T), lambda b: (b, 0, 0)),
            pl.BlockSpec((1, T, E), lambda b: (b, 0, 0)),
            pl.BlockSpec((3 * ch, E), lambda b: (0, 0)),
        ],
        out_specs=pl.BlockSpec((1, ch, T), lambda b: (b, 0, 0)),
        out_shape=jax.ShapeDtypeStruct((B, ch, T), qkv.dtype),
    )(q, kv, cls_embedding, W_cls)
    return out.reshape(bs, n_heads * ch, T)
